# baseline (speedup 1.0000x reference)
import functools
import jax
import jax.numpy as jnp
from jax import lax
from jax.experimental import pallas as pl
from jax.experimental.pallas import tpu as pltpu
from jax.experimental.pallas import tpu_sc as plsc

_B, _S, _D = 4096, 50, 64
_N = _B * _S
_NW = 32
_PER_W = _N // _NW
_CHUNK = 128
_NCH = _PER_W // _CHUNK
_SCALE = 8.0
_LANES = 16
_NBUF = 2


def _body(tok_hbm, w_hbm, out_hbm, idx_v, ibufs, obufs, gsems, ssems):
    c = lax.axis_index("c")
    s = lax.axis_index("s")
    wid = s * 2 + c
    pltpu.sync_copy(tok_hbm.at[wid], idx_v)

    def start_gather(cg, b):
        pltpu.make_async_copy(w_hbm.at[idx_v.at[cg]], ibufs[b], gsems[b]).start()

    def scale(b):
        def row_body(r, _):
            for j in range(_D // _LANES):
                sl = pl.ds(j * _LANES, _LANES)
                obufs[b][r, sl] = ibufs[b][r, sl] * _SCALE
            return 0

        lax.fori_loop(0, _CHUNK, row_body, 0, unroll=8)

    for b in range(_NBUF):
        start_gather(b, b)

    def outer(g, _):
        for b in range(_NBUF):
            cg = g * _NBUF + b
            pltpu.make_async_copy(w_hbm.at[idx_v.at[cg]], ibufs[b], gsems[b]).wait()

            @pl.when(g > 0)
            def _():
                pltpu.make_async_copy(
                    obufs[b], out_hbm.at[pl.ds(0, _CHUNK)], ssems[b]
                ).wait()

            scale(b)

            @pl.when(g < (_NCH // _NBUF) - 1)
            def _():
                start_gather(cg + _NBUF, b)

            base = wid * _PER_W + cg * _CHUNK
            pltpu.make_async_copy(
                obufs[b], out_hbm.at[pl.ds(base, _CHUNK)], ssems[b]
            ).start()
        return 0

    lax.fori_loop(0, _NCH // _NBUF, outer, 0)
    for b in range(_NBUF):
        pltpu.make_async_copy(obufs[b], out_hbm.at[pl.ds(0, _CHUNK)], ssems[b]).wait()


_launch = functools.partial(
    pl.kernel,
    out_type=jax.ShapeDtypeStruct((_N, _D), jnp.float32),
    mesh=plsc.VectorSubcoreMesh(core_axis_name="c", subcore_axis_name="s"),
    scratch_types=[
        pltpu.VMEM((_NCH, _CHUNK), jnp.int32),
        [pltpu.VMEM((_CHUNK, _D), jnp.float32) for _ in range(_NBUF)],
        [pltpu.VMEM((_CHUNK, _D), jnp.float32) for _ in range(_NBUF)],
        [pltpu.SemaphoreType.DMA for _ in range(_NBUF)],
        [pltpu.SemaphoreType.DMA for _ in range(_NBUF)],
    ],
    compiler_params=pltpu.CompilerParams(use_tc_tiling_on_sc=False),
)(_body)


def kernel(tokens, W):
    tok = tokens.reshape(_NW, _NCH, _CHUNK)
    out = _launch(tok, W)
    return out.reshape(_B, _S, _D)
